# baseline (speedup 1.0000x reference)
import jax
import jax.numpy as jnp
from jax.experimental import pallas as pl
from jax.experimental.pallas import tpu as pltpu


def _gcn_kernel(scal_ref, a_ref, f_ref, h0_ref, w_ref, out_ref):
    alpha = scal_ref[0]
    theta = scal_ref[1]
    agg = jnp.dot(a_ref[...].astype(jnp.bfloat16),
                  f_ref[...].astype(jnp.bfloat16),
                  preferred_element_type=jnp.float32)
    support = (1.0 - alpha) * agg + alpha * h0_ref[...]
    lin = jnp.dot(support, w_ref[...], preferred_element_type=jnp.float32)
    out = theta * lin + (1.0 - theta) * support
    out_ref[...] = jnp.maximum(out, 0.0)


def kernel(features, A, h0, W, lamda, alpha, l):
    B, N, D = features.shape
    theta = jnp.log(lamda / l + 1.0)
    scal = jnp.stack([jnp.float32(alpha), jnp.float32(theta)])
    f2 = features.reshape(N, D)
    h2 = h0.reshape(N, D)

    RB = 200
    out = pl.pallas_call(
        _gcn_kernel,
        grid=(N // RB,),
        in_specs=[
            pl.BlockSpec(memory_space=pltpu.SMEM),
            pl.BlockSpec((RB, N), lambda i: (i, 0)),
            pl.BlockSpec((N, D), lambda i: (0, 0)),
            pl.BlockSpec((RB, D), lambda i: (i, 0)),
            pl.BlockSpec((D, D), lambda i: (0, 0)),
        ],
        out_specs=pl.BlockSpec((RB, D), lambda i: (i, 0)),
        out_shape=jax.ShapeDtypeStruct((N, D), jnp.float32),
    )(scal, A, f2, h2, W)
    return out.reshape(B, N, D)


# speedup vs baseline: 1.0132x; 1.0132x over previous
import jax
import jax.numpy as jnp
from jax.experimental import pallas as pl
from jax.experimental.pallas import tpu as pltpu


def _gcn_kernel(scal_ref, a_ref, f_ref, h0_ref, w_ref, out_ref):
    alpha = scal_ref[0]
    theta = scal_ref[1]
    agg = jnp.dot(a_ref[...], f_ref[...], preferred_element_type=jnp.float32)
    support = (1.0 - alpha) * agg + alpha * h0_ref[...]
    lin = jnp.dot(support, w_ref[...], preferred_element_type=jnp.float32)
    out = theta * lin + (1.0 - theta) * support
    out_ref[...] = jnp.maximum(out, 0.0)


def kernel(features, A, h0, W, lamda, alpha, l):
    B, N, D = features.shape
    theta = jnp.log(lamda / l + 1.0)
    scal = jnp.stack([jnp.float32(alpha), jnp.float32(theta)])
    f2 = features.reshape(N, D)
    h2 = h0.reshape(N, D)

    RB = 400
    out = pl.pallas_call(
        _gcn_kernel,
        grid=(N // RB,),
        in_specs=[
            pl.BlockSpec(memory_space=pltpu.SMEM),
            pl.BlockSpec((RB, N), lambda i: (i, 0)),
            pl.BlockSpec((N, D), lambda i: (0, 0)),
            pl.BlockSpec((RB, D), lambda i: (i, 0)),
            pl.BlockSpec((D, D), lambda i: (0, 0)),
        ],
        out_specs=pl.BlockSpec((RB, D), lambda i: (i, 0)),
        out_shape=jax.ShapeDtypeStruct((N, D), jnp.float32),
    )(scal, A, f2, h2, W)
    return out.reshape(B, N, D)
